# baseline (speedup 1.0000x reference)
import functools

import jax
import jax.numpy as jnp
from jax import lax
from jax.experimental import pallas as pl
from jax.experimental.pallas import tpu as pltpu
from jax.experimental.pallas import tpu_sc as plsc

NC = 2
NS = 16
NW = NC * NS
D = 128
DW = 144
CH = 128
N_OUT = 1024


def _make_agg(ndst_pad, ept, filter_lt=None, nrep=1):
  nch_max = ept // CH
  assert ept % CH == 0
  rows_z = nrep * ndst_pad // NS
  rows_o = N_OUT // NS
  cap = ept + 2 * CH
  mesh = plsc.VectorSubcoreMesh(
      core_axis_name="c", subcore_axis_name="s", num_cores=NC, num_subcores=NS)

  scratch = [
      pltpu.VMEM_SHARED((nrep * ndst_pad, DW), jnp.float32),
      pltpu.VMEM((ept,), jnp.int32),
      pltpu.VMEM((ept,), jnp.int32),
      [pltpu.VMEM((CH, DW), jnp.float32) for _ in range(2)],
      pltpu.SemaphoreType.DMA,
      pltpu.SemaphoreType.DMA,
  ]
  if filter_lt is not None:
    scratch += [pltpu.VMEM((cap,), jnp.int32), pltpu.VMEM((cap,), jnp.int32)]

  @functools.partial(
      pl.kernel,
      mesh=mesh,
      out_type=jax.ShapeDtypeStruct((NC * nrep, N_OUT, DW), jnp.float32),
      compiler_params=pltpu.CompilerParams(
          use_tc_tiling_on_sc=False, needs_layout_passes=False),
      scratch_types=tuple(scratch),
  )
  def agg(table_hbm, src_hbm, dst_hbm, zrow_hbm,
          acc_out, acc_sh, raw_src, raw_dst, rows_v, sem_g, sem_s, *comp):
    c = lax.axis_index("c")
    s = lax.axis_index("s")
    wid = s * NC + c
    roff = (s % nrep) * ndst_pad

    pltpu.sync_copy(zrow_hbm.at[pl.ds(0, rows_z)],
                    acc_sh.at[pl.ds(s * rows_z, rows_z)])

    base0 = wid * ept
    pltpu.sync_copy(src_hbm.at[pl.ds(base0, ept)], raw_src)
    pltpu.sync_copy(dst_hbm.at[pl.ds(base0, ept)], raw_dst)

    if filter_lt is not None:
      csrc, cdst = comp

      def comp_body(j, off):
        sv = raw_src[pl.ds(j * 16, 16)]
        dv = raw_dst[pl.ds(j * 16, 16)]
        mask = dv < filter_lt
        cum = plsc.cumsum(jnp.where(mask, 1, 0).astype(jnp.int32))
        pos = off + cum - 1
        plsc.store_scatter(csrc, [pos], sv, mask=mask)
        plsc.store_scatter(cdst, [pos], dv + roff, mask=mask)
        return off + jnp.max(cum)

      n = lax.fori_loop(0, ept // 16, comp_body, 0)
      dummy_s = jnp.zeros((16,), jnp.int32)
      dummy_d = jnp.full((16,), ndst_pad - 8, jnp.int32) + roff
      for t in range(2 * CH // 16):
        csrc[pl.ds(n + t * 16, 16)] = dummy_s
        cdst[pl.ds(n + t * 16, 16)] = dummy_d
      nr = jnp.maximum(lax.shift_right_logical(n + (2 * CH - 1), 8), 1)
      src_idx, dst_idx = csrc, cdst
    else:
      if nrep > 1:
        def roff_body(j, carry):
          raw_dst[pl.ds(j * 16, 16)] = raw_dst[pl.ds(j * 16, 16)] + roff
          return carry

        lax.fori_loop(0, ept // 16, roff_body, 0)
      nr = nch_max // 2
      src_idx, dst_idx = raw_src, raw_dst

    def body(r, carry):
      b = r * (2 * CH)
      for k in range(2):
        pltpu.async_copy(table_hbm.at[src_idx.at[pl.ds(b + k * CH, CH)]],
                         rows_v[k], sem_g).wait()
        pltpu.sync_copy(rows_v[k],
                        acc_sh.at[dst_idx.at[pl.ds(b + k * CH, CH)]],
                        add=True)
      return carry

    lax.fori_loop(0, nr, body, 0)
    plsc.subcore_barrier()

    for rep in range(nrep):
      pltpu.sync_copy(
          acc_sh.at[pl.ds(rep * ndst_pad + s * rows_o, rows_o)],
          acc_out.at[c * nrep + rep, pl.ds(s * rows_o, rows_o)])

  return agg


def _augment(table):
  n = table.shape[0]
  return jnp.concatenate(
      [table, jnp.ones((n, 1), jnp.float32), jnp.zeros((n, DW - D - 1), jnp.float32)],
      axis=1)


def _sage_update(acc, x_dst, W_l, b_l, W_r):
  def body(acc_ref, xt_ref, wl_ref, bl_ref, wr_ref, o_ref):
    nparts = acc_ref.shape[0]
    ssum = sum(acc_ref[i][:, :D] for i in range(nparts))
    csum = sum(acc_ref[i][:, D:D + 1] for i in range(nparts))
    mean = ssum / jnp.maximum(csum, 1.0)
    t1 = lax.dot_general(mean, wl_ref[...], (((1,), (1,)), ((), ())),
                         preferred_element_type=jnp.float32)
    t2 = lax.dot_general(xt_ref[...], wr_ref[...], (((1,), (1,)), ((), ())),
                         preferred_element_type=jnp.float32)
    o_ref[...] = jnp.maximum(t1 + t2 + bl_ref[...], 0.0)

  return pl.pallas_call(
      body,
      out_shape=jax.ShapeDtypeStruct((N_OUT, D), jnp.float32),
  )(acc, x_dst, W_l, b_l.reshape(1, D), W_r)


def _pad_edges(edge_index, e_pad, dummy_dst, ept, n_table):
  src = edge_index[0].astype(jnp.int32)
  dst = edge_index[1].astype(jnp.int32)
  n = e_pad - src.shape[0]
  src = jnp.concatenate([src, jnp.zeros((n,), jnp.int32)])
  dst = jnp.concatenate([dst, jnp.full((n,), dummy_dst, jnp.int32)])
  off = (jnp.arange(NW, dtype=jnp.int32) % NC) * n_table
  src = (src.reshape(NW, ept) + off[:, None]).reshape(-1)
  return src, dst


def kernel(x, edge_index1, edge_index2, n_target1, n_target2,
           W_l1, b_l1, W_r1, W_l2, b_l2, W_r2):
  ND1, EPT1 = 1152, 10240
  ND2, EPT2 = 1152, 2048

  N_SRC1 = 5000
  src1, dst1 = _pad_edges(edge_index1, NW * EPT1, ND1 - 8, EPT1, N_SRC1)
  src2, dst2 = _pad_edges(edge_index2, NW * EPT2, ND2 - 8, EPT2, N_OUT)

  zrow = jnp.zeros((2 * ND1 // NS, DW), jnp.float32)

  t1 = _augment(x[:N_SRC1])
  agg1 = _make_agg(ND1, EPT1, filter_lt=N_OUT)
  acc1 = agg1(jnp.concatenate([t1, t1]), src1, dst1, zrow)
  h1 = _sage_update(acc1, x[:N_OUT], W_l1, b_l1, W_r1)

  t2 = _augment(h1)
  agg2 = _make_agg(ND2, EPT2)
  acc2 = agg2(jnp.concatenate([t2, t2]), src2, dst2, zrow)
  out = _sage_update(acc2, h1, W_l2, b_l2, W_r2)
  return out


# speedup vs baseline: 1.1466x; 1.1466x over previous
import functools

import jax
import jax.numpy as jnp
from jax import lax
from jax.experimental import pallas as pl
from jax.experimental.pallas import tpu as pltpu
from jax.experimental.pallas import tpu_sc as plsc

NC = 2
NS = 16
NW = NC * NS
D = 128
DW = 144
CH = 128
N_OUT = 1024


def _make_agg(ndst_pad, ept, filter_lt=None, nrep=1):
  nch_max = ept // CH
  assert ept % CH == 0
  rows_z = nrep * ndst_pad // NS
  rows_o = N_OUT // NS
  cap = ept + 2 * CH
  mesh = plsc.VectorSubcoreMesh(
      core_axis_name="c", subcore_axis_name="s", num_cores=NC, num_subcores=NS)

  scratch = [
      pltpu.VMEM_SHARED((nrep * ndst_pad, DW), jnp.float32),
      pltpu.VMEM((ept,), jnp.int32),
      pltpu.VMEM((ept,), jnp.int32),
      [pltpu.VMEM((CH, DW), jnp.float32) for _ in range(2)],
      pltpu.SemaphoreType.DMA,
      pltpu.SemaphoreType.DMA,
  ]
  if filter_lt is not None:
    scratch += [pltpu.VMEM((cap,), jnp.int32), pltpu.VMEM((cap,), jnp.int32)]

  @functools.partial(
      pl.kernel,
      mesh=mesh,
      out_type=jax.ShapeDtypeStruct((NC * nrep, N_OUT, DW), jnp.float32),
      compiler_params=pltpu.CompilerParams(
          use_tc_tiling_on_sc=False, needs_layout_passes=False),
      scratch_types=tuple(scratch),
  )
  def agg(table_hbm, src_hbm, dst_hbm, zrow_hbm,
          acc_out, acc_sh, raw_src, raw_dst, rows_v, sem_g, sem_s, *comp):
    c = lax.axis_index("c")
    s = lax.axis_index("s")
    wid = s * NC + c
    roff = (s % nrep) * ndst_pad

    pltpu.sync_copy(zrow_hbm.at[pl.ds(0, rows_z)],
                    acc_sh.at[pl.ds(s * rows_z, rows_z)])

    base0 = wid * ept
    pltpu.sync_copy(src_hbm.at[pl.ds(base0, ept)], raw_src)
    pltpu.sync_copy(dst_hbm.at[pl.ds(base0, ept)], raw_dst)

    if filter_lt is not None:
      csrc, cdst = comp

      def comp_body(j, off):
        sv = raw_src[pl.ds(j * 16, 16)]
        dv = raw_dst[pl.ds(j * 16, 16)]
        mask = dv < filter_lt
        cum = plsc.cumsum(jnp.where(mask, 1, 0).astype(jnp.int32))
        pos = off + cum - 1
        plsc.store_scatter(csrc, [pos], sv, mask=mask)
        plsc.store_scatter(cdst, [pos], dv + roff, mask=mask)
        return off + jnp.max(cum)

      n = lax.fori_loop(0, ept // 16, comp_body, 0)
      dummy_s = jnp.zeros((16,), jnp.int32)
      dummy_d = jnp.full((16,), ndst_pad - 8, jnp.int32) + roff
      for t in range(2 * CH // 16):
        csrc[pl.ds(n + t * 16, 16)] = dummy_s
        cdst[pl.ds(n + t * 16, 16)] = dummy_d
      nr = jnp.maximum(lax.shift_right_logical(n + (2 * CH - 1), 8), 1)
      src_idx, dst_idx = csrc, cdst
    else:
      if nrep > 1:
        def roff_body(j, carry):
          raw_dst[pl.ds(j * 16, 16)] = raw_dst[pl.ds(j * 16, 16)] + roff
          return carry

        lax.fori_loop(0, ept // 16, roff_body, 0)
      nr = nch_max // 2
      src_idx, dst_idx = raw_src, raw_dst

    def body(r, carry):
      b = r * (2 * CH)
      for k in range(2):
        pltpu.async_copy(table_hbm.at[src_idx.at[pl.ds(b + k * CH, CH)]],
                         rows_v[k], sem_g).wait()
        pltpu.sync_copy(rows_v[k],
                        acc_sh.at[dst_idx.at[pl.ds(b + k * CH, CH)]],
                        add=True)
      return carry

    lax.fori_loop(0, nr, body, 0)
    plsc.subcore_barrier()

    for rep in range(nrep):
      pltpu.sync_copy(
          acc_sh.at[pl.ds(rep * ndst_pad + s * rows_o, rows_o)],
          acc_out.at[c * nrep + rep, pl.ds(s * rows_o, rows_o)])

  return agg


def _augment(table):
  n = table.shape[0]
  return jnp.concatenate(
      [table, jnp.ones((n, 1), jnp.float32), jnp.zeros((n, DW - D - 1), jnp.float32)],
      axis=1)


def _sage_update(acc, x_dst, W_l, b_l, W_r):
  def body(acc_ref, xt_ref, wl_ref, bl_ref, wr_ref, o_ref):
    nparts = acc_ref.shape[0]
    ssum = sum(acc_ref[i][:, :D] for i in range(nparts))
    csum = sum(acc_ref[i][:, D:D + 1] for i in range(nparts))
    mean = ssum / jnp.maximum(csum, 1.0)
    t1 = lax.dot_general(mean, wl_ref[...], (((1,), (1,)), ((), ())),
                         preferred_element_type=jnp.float32)
    t2 = lax.dot_general(xt_ref[...], wr_ref[...], (((1,), (1,)), ((), ())),
                         preferred_element_type=jnp.float32)
    o_ref[...] = jnp.maximum(t1 + t2 + bl_ref[...], 0.0)

  return pl.pallas_call(
      body,
      out_shape=jax.ShapeDtypeStruct((N_OUT, D), jnp.float32),
  )(acc, x_dst, W_l, b_l.reshape(1, D), W_r)


def _pad_edges(edge_index, e_pad, dummy_dst, ept, n_table):
  src = edge_index[0].astype(jnp.int32)
  dst = edge_index[1].astype(jnp.int32)
  n = e_pad - src.shape[0]
  src = jnp.concatenate([src, jnp.zeros((n,), jnp.int32)])
  dst = jnp.concatenate([dst, jnp.full((n,), dummy_dst, jnp.int32)])
  return src, dst


def kernel(x, edge_index1, edge_index2, n_target1, n_target2,
           W_l1, b_l1, W_r1, W_l2, b_l2, W_r2):
  ND1, EPT1 = 1152, 10240
  ND2, EPT2 = 1152, 2048

  N_SRC1 = 5000
  src1, dst1 = _pad_edges(edge_index1, NW * EPT1, ND1 - 8, EPT1, N_SRC1)
  src2, dst2 = _pad_edges(edge_index2, NW * EPT2, ND2 - 8, EPT2, N_OUT)

  zrow = jnp.zeros((2 * ND1 // NS, DW), jnp.float32)

  agg1 = _make_agg(ND1, EPT1, filter_lt=N_OUT)
  acc1 = agg1(_augment(x[:N_SRC1]), src1, dst1, zrow)
  h1 = _sage_update(acc1, x[:N_OUT], W_l1, b_l1, W_r1)

  agg2 = _make_agg(ND2, EPT2)
  acc2 = agg2(_augment(h1), src2, dst2, zrow)
  out = _sage_update(acc2, h1, W_l2, b_l2, W_r2)
  return out


# speedup vs baseline: 1.6518x; 1.4406x over previous
import functools

import jax
import jax.numpy as jnp
from jax import lax
from jax.experimental import pallas as pl
from jax.experimental.pallas import tpu as pltpu
from jax.experimental.pallas import tpu_sc as plsc

NC = 2
NS = 16
NW = NC * NS
D = 128
DW = 144
CH = 128
N_OUT = 1024


def _make_agg(ndst_pad, ept, filter_lt=None, nrep=1):
  nch_max = ept // CH
  assert ept % CH == 0
  rows_z = nrep * ndst_pad // NS
  rows_o = N_OUT // NS
  cap = ept + CH
  mesh = plsc.VectorSubcoreMesh(
      core_axis_name="c", subcore_axis_name="s", num_cores=NC, num_subcores=NS)

  scratch = [
      pltpu.VMEM_SHARED((nrep * ndst_pad, DW), jnp.float32),
      pltpu.VMEM((ept,), jnp.int32),
      pltpu.VMEM((ept,), jnp.int32),
      pltpu.VMEM((CH, DW), jnp.float32),
      pltpu.SemaphoreType.DMA,
  ]
  if filter_lt is not None:
    scratch += [pltpu.VMEM((cap,), jnp.int32), pltpu.VMEM((cap,), jnp.int32)]

  @functools.partial(
      pl.kernel,
      mesh=mesh,
      out_type=jax.ShapeDtypeStruct((NC * nrep, N_OUT, DW), jnp.float32),
      compiler_params=pltpu.CompilerParams(
          use_tc_tiling_on_sc=False, needs_layout_passes=False),
      scratch_types=tuple(scratch),
  )
  def agg(table_hbm, src_hbm, dst_hbm, zrow_hbm,
          acc_out, acc_sh, raw_src, raw_dst, rows_v, sem, *comp):
    c = lax.axis_index("c")
    s = lax.axis_index("s")
    wid = s * NC + c
    roff = (s % nrep) * ndst_pad

    pltpu.sync_copy(zrow_hbm.at[pl.ds(0, rows_z)],
                    acc_sh.at[pl.ds(s * rows_z, rows_z)])

    base0 = wid * ept
    pltpu.sync_copy(src_hbm.at[pl.ds(base0, ept)], raw_src)
    pltpu.sync_copy(dst_hbm.at[pl.ds(base0, ept)], raw_dst)

    if filter_lt is not None:
      csrc, cdst = comp

      def comp_body(j, off):
        sv = raw_src[pl.ds(j * 16, 16)]
        dv = raw_dst[pl.ds(j * 16, 16)]
        mask = dv < filter_lt
        cum = plsc.cumsum(jnp.where(mask, 1, 0).astype(jnp.int32))
        pos = off + cum - 1
        plsc.store_scatter(csrc, [pos], sv, mask=mask)
        plsc.store_scatter(cdst, [pos], dv + roff, mask=mask)
        return off + jnp.max(cum)

      n = lax.fori_loop(0, ept // 16, comp_body, 0)
      dummy_s = jnp.zeros((16,), jnp.int32)
      dummy_d = jnp.full((16,), ndst_pad - 8, jnp.int32) + roff
      for t in range(CH // 16):
        csrc[pl.ds(n + t * 16, 16)] = dummy_s
        cdst[pl.ds(n + t * 16, 16)] = dummy_d
      nr = lax.shift_right_logical(n + (CH - 1), 7)
      src_idx, dst_idx = csrc, cdst
    else:
      if nrep > 1:
        def roff_body(j, carry):
          raw_dst[pl.ds(j * 16, 16)] = raw_dst[pl.ds(j * 16, 16)] + roff
          return carry

        lax.fori_loop(0, ept // 16, roff_body, 0)
      nr = nch_max
      src_idx, dst_idx = raw_src, raw_dst

    def body(g, carry):
      b = g * CH
      pltpu.async_copy(table_hbm.at[src_idx.at[pl.ds(b, CH)]], rows_v,
                       sem).wait()
      pltpu.sync_copy(rows_v, acc_sh.at[dst_idx.at[pl.ds(b, CH)]], add=True)
      return carry

    lax.fori_loop(0, nr, body, 0)
    plsc.subcore_barrier()

    for rep in range(nrep):
      pltpu.sync_copy(
          acc_sh.at[pl.ds(rep * ndst_pad + s * rows_o, rows_o)],
          acc_out.at[c * nrep + rep, pl.ds(s * rows_o, rows_o)])

  return agg


def _augment(table):
  n = table.shape[0]
  return jnp.concatenate(
      [table, jnp.ones((n, 1), jnp.float32), jnp.zeros((n, DW - D - 1), jnp.float32)],
      axis=1)


def _sage_update(acc, x_dst, W_l, b_l, W_r):
  def body(acc_ref, xt_ref, wl_ref, bl_ref, wr_ref, o_ref):
    nparts = acc_ref.shape[0]
    ssum = sum(acc_ref[i][:, :D] for i in range(nparts))
    csum = sum(acc_ref[i][:, D:D + 1] for i in range(nparts))
    mean = ssum / jnp.maximum(csum, 1.0)
    t1 = lax.dot_general(mean, wl_ref[...], (((1,), (1,)), ((), ())),
                         preferred_element_type=jnp.float32)
    t2 = lax.dot_general(xt_ref[...], wr_ref[...], (((1,), (1,)), ((), ())),
                         preferred_element_type=jnp.float32)
    o_ref[...] = jnp.maximum(t1 + t2 + bl_ref[...], 0.0)

  return pl.pallas_call(
      body,
      out_shape=jax.ShapeDtypeStruct((N_OUT, D), jnp.float32),
  )(acc, x_dst, W_l, b_l.reshape(1, D), W_r)


def _pad_edges(edge_index, e_pad, dummy_dst, ept, n_table):
  src = edge_index[0].astype(jnp.int32)
  dst = edge_index[1].astype(jnp.int32)
  n = e_pad - src.shape[0]
  src = jnp.concatenate([src, jnp.zeros((n,), jnp.int32)])
  dst = jnp.concatenate([dst, jnp.full((n,), dummy_dst, jnp.int32)])
  return src, dst


def kernel(x, edge_index1, edge_index2, n_target1, n_target2,
           W_l1, b_l1, W_r1, W_l2, b_l2, W_r2):
  ND1, EPT1 = 1152, 10240
  ND2, EPT2 = 1152, 2048

  N_SRC1 = 5000
  src1, dst1 = _pad_edges(edge_index1, NW * EPT1, ND1 - 8, EPT1, N_SRC1)
  src2, dst2 = _pad_edges(edge_index2, NW * EPT2, ND2 - 8, EPT2, N_OUT)

  zrow = jnp.zeros((2 * ND1 // NS, DW), jnp.float32)

  agg1 = _make_agg(ND1, EPT1, filter_lt=N_OUT)
  acc1 = agg1(_augment(x), src1, dst1, zrow)
  h1 = _sage_update(acc1, x[:N_OUT], W_l1, b_l1, W_r1)

  agg2 = _make_agg(ND2, EPT2)
  acc2 = agg2(_augment(h1), src2, dst2, zrow)
  out = _sage_update(acc2, h1, W_l2, b_l2, W_r2)
  return out


# speedup vs baseline: 1.6627x; 1.0066x over previous
import functools

import jax
import jax.numpy as jnp
from jax import lax
from jax.experimental import pallas as pl
from jax.experimental.pallas import tpu as pltpu
from jax.experimental.pallas import tpu_sc as plsc

NC = 2
NS = 16
NW = NC * NS
D = 128
DW = 144
CH = 128
N_OUT = 1024


def _make_agg(ndst_pad, ept, filter_lt=None, nrep=1):
  nch_max = ept // CH
  assert ept % CH == 0
  rows_z = nrep * ndst_pad // NS
  rows_o = N_OUT // NS
  cap = ept + CH
  mesh = plsc.VectorSubcoreMesh(
      core_axis_name="c", subcore_axis_name="s", num_cores=NC, num_subcores=NS)

  scratch = [
      pltpu.VMEM_SHARED((nrep * ndst_pad, DW), jnp.float32),
      pltpu.VMEM((ept,), jnp.int32),
      pltpu.VMEM((ept,), jnp.int32),
      pltpu.VMEM((CH, DW), jnp.float32),
      pltpu.SemaphoreType.DMA,
  ]
  if filter_lt is not None:
    scratch += [pltpu.VMEM((cap,), jnp.int32), pltpu.VMEM((cap,), jnp.int32)]

  @functools.partial(
      pl.kernel,
      mesh=mesh,
      out_type=jax.ShapeDtypeStruct((NC * nrep, N_OUT, DW), jnp.float32),
      compiler_params=pltpu.CompilerParams(
          use_tc_tiling_on_sc=False, needs_layout_passes=False),
      scratch_types=tuple(scratch),
  )
  def agg(table_hbm, src_hbm, dst_hbm, zrow_hbm,
          acc_out, acc_sh, raw_src, raw_dst, rows_v, sem, *comp):
    c = lax.axis_index("c")
    s = lax.axis_index("s")
    wid = s * NC + c
    roff = (s % nrep) * ndst_pad if nrep > 1 else 0

    pltpu.sync_copy(zrow_hbm.at[pl.ds(0, rows_z)],
                    acc_sh.at[pl.ds(s * rows_z, rows_z)])

    base0 = wid * ept
    pltpu.sync_copy(src_hbm.at[pl.ds(base0, ept)], raw_src)
    pltpu.sync_copy(dst_hbm.at[pl.ds(base0, ept)], raw_dst)

    if filter_lt is not None:
      csrc, cdst = comp

      def comp_body(j, off):
        for u in range(2):
          b = j * 32 + u * 16
          sv = raw_src[pl.ds(b, 16)]
          dv = raw_dst[pl.ds(b, 16)]
          mask = dv < filter_lt
          cum = plsc.cumsum(mask.astype(jnp.int32))
          pos = off + cum - 1
          plsc.store_scatter(csrc, [pos], sv, mask=mask)
          if nrep > 1:
            dv = dv + roff
          plsc.store_scatter(cdst, [pos], dv, mask=mask)
          off = off + cum[15]
        return off

      n = lax.fori_loop(0, ept // 32, comp_body, 0)
      dummy_s = jnp.zeros((16,), jnp.int32)
      dummy_d = jnp.full((16,), ndst_pad - 8, jnp.int32) + roff
      for t in range(CH // 16):
        csrc[pl.ds(n + t * 16, 16)] = dummy_s
        cdst[pl.ds(n + t * 16, 16)] = dummy_d
      nr = lax.shift_right_logical(n + (CH - 1), 7)
      src_idx, dst_idx = csrc, cdst
    else:
      if nrep > 1:
        def roff_body(j, carry):
          raw_dst[pl.ds(j * 16, 16)] = raw_dst[pl.ds(j * 16, 16)] + roff
          return carry

        lax.fori_loop(0, ept // 16, roff_body, 0)
      nr = nch_max
      src_idx, dst_idx = raw_src, raw_dst

    def body(g, carry):
      b = g * CH
      pltpu.async_copy(table_hbm.at[src_idx.at[pl.ds(b, CH)]], rows_v,
                       sem).wait()
      pltpu.sync_copy(rows_v, acc_sh.at[dst_idx.at[pl.ds(b, CH)]], add=True)
      return carry

    lax.fori_loop(0, nr, body, 0)
    plsc.subcore_barrier()

    for rep in range(nrep):
      pltpu.sync_copy(
          acc_sh.at[pl.ds(rep * ndst_pad + s * rows_o, rows_o)],
          acc_out.at[c * nrep + rep, pl.ds(s * rows_o, rows_o)])

  return agg


def _augment(table):
  n = table.shape[0]
  return jnp.concatenate(
      [table, jnp.ones((n, 1), jnp.float32), jnp.zeros((n, DW - D - 1), jnp.float32)],
      axis=1)


def _sage_update(acc, x_dst, W_l, b_l, W_r):
  def body(acc_ref, xt_ref, wl_ref, bl_ref, wr_ref, o_ref):
    nparts = acc_ref.shape[0]
    ssum = sum(acc_ref[i][:, :D] for i in range(nparts))
    csum = sum(acc_ref[i][:, D:D + 1] for i in range(nparts))
    mean = ssum / jnp.maximum(csum, 1.0)
    t1 = lax.dot_general(mean, wl_ref[...], (((1,), (1,)), ((), ())),
                         preferred_element_type=jnp.float32)
    t2 = lax.dot_general(xt_ref[...], wr_ref[...], (((1,), (1,)), ((), ())),
                         preferred_element_type=jnp.float32)
    o_ref[...] = jnp.maximum(t1 + t2 + bl_ref[...], 0.0)

  return pl.pallas_call(
      body,
      out_shape=jax.ShapeDtypeStruct((N_OUT, D), jnp.float32),
  )(acc, x_dst, W_l, b_l.reshape(1, D), W_r)


def _pad_edges(edge_index, e_pad, dummy_dst, ept, n_table):
  src = edge_index[0].astype(jnp.int32)
  dst = edge_index[1].astype(jnp.int32)
  n = e_pad - src.shape[0]
  src = jnp.concatenate([src, jnp.zeros((n,), jnp.int32)])
  dst = jnp.concatenate([dst, jnp.full((n,), dummy_dst, jnp.int32)])
  return src, dst


def kernel(x, edge_index1, edge_index2, n_target1, n_target2,
           W_l1, b_l1, W_r1, W_l2, b_l2, W_r2):
  ND1, EPT1 = 1152, 10240
  ND2, EPT2 = 1152, 2048

  N_SRC1 = 5000
  src1, dst1 = _pad_edges(edge_index1, NW * EPT1, ND1 - 8, EPT1, N_SRC1)
  src2, dst2 = _pad_edges(edge_index2, NW * EPT2, ND2 - 8, EPT2, N_OUT)

  zrow = jnp.zeros((2 * ND1 // NS, DW), jnp.float32)

  agg1 = _make_agg(ND1, EPT1, filter_lt=N_OUT)
  acc1 = agg1(_augment(x), src1, dst1, zrow)
  h1 = _sage_update(acc1, x[:N_OUT], W_l1, b_l1, W_r1)

  agg2 = _make_agg(ND2, EPT2)
  acc2 = agg2(_augment(h1), src2, dst2, zrow)
  out = _sage_update(acc2, h1, W_l2, b_l2, W_r2)
  return out


# speedup vs baseline: 1.6630x; 1.0002x over previous
import functools

import jax
import jax.numpy as jnp
from jax import lax
from jax.experimental import pallas as pl
from jax.experimental.pallas import tpu as pltpu
from jax.experimental.pallas import tpu_sc as plsc

NC = 2
NS = 16
NW = NC * NS
D = 128
DW = 144
CH = 128
N_OUT = 1024


def _make_agg(ndst_pad, ept, filter_lt=None, nrep=1):
  nch_max = ept // CH
  assert ept % CH == 0
  rows_z = nrep * ndst_pad // NS
  rows_o = N_OUT // NS
  cap = ept + CH
  mesh = plsc.VectorSubcoreMesh(
      core_axis_name="c", subcore_axis_name="s", num_cores=NC, num_subcores=NS)

  scratch = [
      pltpu.VMEM_SHARED((nrep * ndst_pad, DW), jnp.float32),
      pltpu.VMEM((ept,), jnp.int32),
      pltpu.VMEM((ept,), jnp.int32),
      pltpu.VMEM((CH, DW), jnp.float32),
      pltpu.SemaphoreType.DMA,
  ]
  if filter_lt is not None:
    scratch += [pltpu.VMEM((cap,), jnp.int32), pltpu.VMEM((cap,), jnp.int32)]

  @functools.partial(
      pl.kernel,
      mesh=mesh,
      out_type=jax.ShapeDtypeStruct((NC * nrep, N_OUT, DW), jnp.float32),
      compiler_params=pltpu.CompilerParams(
          use_tc_tiling_on_sc=False, needs_layout_passes=False),
      scratch_types=tuple(scratch),
  )
  def agg(table_hbm, src_hbm, dst_hbm, zrow_hbm,
          acc_out, acc_sh, raw_src, raw_dst, rows_v, sem, *comp):
    c = lax.axis_index("c")
    s = lax.axis_index("s")
    wid = s * NC + c
    roff = (s % nrep) * ndst_pad if nrep > 1 else 0

    pltpu.sync_copy(zrow_hbm.at[pl.ds(0, rows_z)],
                    acc_sh.at[pl.ds(s * rows_z, rows_z)])

    base0 = wid * ept
    pltpu.sync_copy(src_hbm.at[pl.ds(base0, ept)], raw_src)
    pltpu.sync_copy(dst_hbm.at[pl.ds(base0, ept)], raw_dst)

    if filter_lt is not None:
      csrc, cdst = comp

      def comp_body(j, off):
        for u in range(4):
          b = j * 64 + u * 16
          sv = raw_src[pl.ds(b, 16)]
          dv = raw_dst[pl.ds(b, 16)]
          mask = dv < filter_lt
          cum = plsc.cumsum(mask.astype(jnp.int32))
          pos = off + cum - 1
          plsc.store_scatter(csrc, [pos], sv, mask=mask)
          if nrep > 1:
            dv = dv + roff
          plsc.store_scatter(cdst, [pos], dv, mask=mask)
          off = off + cum[15]
        return off

      n = lax.fori_loop(0, ept // 64, comp_body, 0)
      dummy_s = jnp.zeros((16,), jnp.int32)
      dummy_d = jnp.full((16,), ndst_pad - 8, jnp.int32) + roff
      for t in range(CH // 16):
        csrc[pl.ds(n + t * 16, 16)] = dummy_s
        cdst[pl.ds(n + t * 16, 16)] = dummy_d
      nr = lax.shift_right_logical(n + (CH - 1), 7)
      src_idx, dst_idx = csrc, cdst
    else:
      if nrep > 1:
        def roff_body(j, carry):
          raw_dst[pl.ds(j * 16, 16)] = raw_dst[pl.ds(j * 16, 16)] + roff
          return carry

        lax.fori_loop(0, ept // 16, roff_body, 0)
      nr = nch_max
      src_idx, dst_idx = raw_src, raw_dst

    def body(g, carry):
      b = g * CH
      pltpu.async_copy(table_hbm.at[src_idx.at[pl.ds(b, CH)]], rows_v,
                       sem).wait()
      pltpu.sync_copy(rows_v, acc_sh.at[dst_idx.at[pl.ds(b, CH)]], add=True)
      return carry

    lax.fori_loop(0, nr, body, 0)
    plsc.subcore_barrier()

    for rep in range(nrep):
      pltpu.sync_copy(
          acc_sh.at[pl.ds(rep * ndst_pad + s * rows_o, rows_o)],
          acc_out.at[c * nrep + rep, pl.ds(s * rows_o, rows_o)])

  return agg


def _augment(table):
  n = table.shape[0]
  return jnp.concatenate(
      [table, jnp.ones((n, 1), jnp.float32), jnp.zeros((n, DW - D - 1), jnp.float32)],
      axis=1)


def _sage_update(acc, x_dst, W_l, b_l, W_r):
  def body(acc_ref, xt_ref, wl_ref, bl_ref, wr_ref, o_ref):
    nparts = acc_ref.shape[0]
    ssum = sum(acc_ref[i][:, :D] for i in range(nparts))
    csum = sum(acc_ref[i][:, D:D + 1] for i in range(nparts))
    mean = ssum / jnp.maximum(csum, 1.0)
    t1 = lax.dot_general(mean, wl_ref[...], (((1,), (1,)), ((), ())),
                         preferred_element_type=jnp.float32)
    t2 = lax.dot_general(xt_ref[...], wr_ref[...], (((1,), (1,)), ((), ())),
                         preferred_element_type=jnp.float32)
    o_ref[...] = jnp.maximum(t1 + t2 + bl_ref[...], 0.0)

  return pl.pallas_call(
      body,
      out_shape=jax.ShapeDtypeStruct((N_OUT, D), jnp.float32),
  )(acc, x_dst, W_l, b_l.reshape(1, D), W_r)


def _pad_edges(edge_index, e_pad, dummy_dst, ept, n_table):
  src = edge_index[0].astype(jnp.int32)
  dst = edge_index[1].astype(jnp.int32)
  n = e_pad - src.shape[0]
  src = jnp.concatenate([src, jnp.zeros((n,), jnp.int32)])
  dst = jnp.concatenate([dst, jnp.full((n,), dummy_dst, jnp.int32)])
  return src, dst


def kernel(x, edge_index1, edge_index2, n_target1, n_target2,
           W_l1, b_l1, W_r1, W_l2, b_l2, W_r2):
  ND1, EPT1 = 1152, 10240
  ND2, EPT2 = 1152, 2048

  N_SRC1 = 5000
  src1, dst1 = _pad_edges(edge_index1, NW * EPT1, ND1 - 8, EPT1, N_SRC1)
  src2, dst2 = _pad_edges(edge_index2, NW * EPT2, ND2 - 8, EPT2, N_OUT)

  zrow = jnp.zeros((2 * ND1 // NS, DW), jnp.float32)

  agg1 = _make_agg(ND1, EPT1, filter_lt=N_OUT)
  acc1 = agg1(_augment(x), src1, dst1, zrow)
  h1 = _sage_update(acc1, x[:N_OUT], W_l1, b_l1, W_r1)

  agg2 = _make_agg(ND2, EPT2)
  acc2 = agg2(_augment(h1), src2, dst2, zrow)
  out = _sage_update(acc2, h1, W_l2, b_l2, W_r2)
  return out
